# Initial kernel scaffold; baseline (speedup 1.0000x reference)
import jax
import jax.numpy as jnp
from jax.experimental import pallas as pl


def kernel(var_c, var_x, con_b, edge_src, edge_dst, edge_weight, W_ve, b_ve, W_ce, b_ce, W1, b1, W2, b2, Wo1, bo1, Wo2, bo2, Wo3, bo3):
    raise NotImplementedError("write your pallas kernel here")


# speedup vs baseline: 24.2825x; 24.2825x over previous
import functools

import jax
import jax.numpy as jnp
from jax import lax
from jax.experimental import pallas as pl
from jax.experimental.pallas import tpu as pltpu
from jax.experimental.pallas import tpu_sc as plsc

NC = 2
NS = 16
NW = NC * NS


def _chunk_size(ew_per):
    for ch in range(min(2000, ew_per) // 16 * 16, 0, -16):
        if ew_per % ch == 0:
            return ch
    for ch in range(min(2000, ew_per), 0, -1):
        if ew_per % ch == 0:
            return ch
    return 1


def _mesh():
    return plsc.VectorSubcoreMesh(
        core_axis_name="c", subcore_axis_name="s", num_cores=NC,
        num_subcores=NS)


def _sc_degrees(edge_src, edge_dst, ones_ch, zeros_n, n_pad):
    e = edge_src.shape[0]
    ew_per = e // NW
    ch = ones_ch.shape[0]
    nchunk = ew_per // ch
    rt = n_pad // NS

    @functools.partial(
        pl.kernel,
        out_type=(
            jax.ShapeDtypeStruct((NC, n_pad), jnp.float32),
            jax.ShapeDtypeStruct((NC, n_pad), jnp.float32),
        ),
        mesh=_mesh(),
        compiler_params=pltpu.CompilerParams(use_tc_tiling_on_sc=False),
        scratch_types=[
            pltpu.VMEM((ch,), jnp.int32),
            pltpu.VMEM((ch,), jnp.float32),
            pltpu.VMEM_SHARED((n_pad,), jnp.float32),
            pltpu.VMEM_SHARED((n_pad,), jnp.float32),
        ],
    )
    def deg_kernel(src_hbm, dst_hbm, ones_hbm, zeros_hbm, dv_out, dc_out,
                   idx_v, ones_v, dv_sh, dc_sh):
        cid = lax.axis_index("c")
        sid = lax.axis_index("s")
        wid = cid * NS + sid
        base = wid * ew_per

        pltpu.sync_copy(zeros_hbm.at[pl.ds(sid * rt, rt)],
                        dv_sh.at[pl.ds(sid * rt, rt)])
        pltpu.sync_copy(zeros_hbm.at[pl.ds(sid * rt, rt)],
                        dc_sh.at[pl.ds(sid * rt, rt)])
        pltpu.sync_copy(ones_hbm, ones_v)
        plsc.subcore_barrier()

        def chunk(j, carry):
            off = base + j * ch
            pltpu.sync_copy(src_hbm.at[pl.ds(off, ch)], idx_v)
            pltpu.sync_copy(ones_v, dv_sh.at[idx_v], add=True)
            pltpu.sync_copy(dst_hbm.at[pl.ds(off, ch)], idx_v)
            pltpu.sync_copy(ones_v, dc_sh.at[idx_v], add=True)
            return carry

        lax.fori_loop(0, nchunk, chunk, 0)
        plsc.subcore_barrier()

        pltpu.sync_copy(dv_sh.at[pl.ds(sid * rt, rt)],
                        dv_out.at[cid, pl.ds(sid * rt, rt)])
        pltpu.sync_copy(dc_sh.at[pl.ds(sid * rt, rt)],
                        dc_out.at[cid, pl.ds(sid * rt, rt)])

    return deg_kernel(edge_src, edge_dst, ones_ch, zeros_n)


def _sc_message_pass(table, gidx, sidx, ew, zeros_r, n_pad):
    e = gidx.shape[0]
    ew_per = e // NW
    ch = _chunk_size(ew_per)
    nchunk = ew_per // ch
    rt = n_pad // NS

    @functools.partial(
        pl.kernel,
        out_type=jax.ShapeDtypeStruct((NC, n_pad, 16), jnp.float32),
        mesh=_mesh(),
        compiler_params=pltpu.CompilerParams(use_tc_tiling_on_sc=False),
        scratch_types=[
            pltpu.VMEM((ch,), jnp.int32),
            pltpu.VMEM((ch,), jnp.int32),
            pltpu.VMEM((ch,), jnp.float32),
            pltpu.VMEM((ch, 16), jnp.float32),
            pltpu.VMEM_SHARED((n_pad, 16), jnp.float32),
            pltpu.SemaphoreType.DMA,
        ],
    )
    def mp_kernel(table_hbm, gidx_hbm, sidx_hbm, ew_hbm, zeros_hbm, agg_out,
                  gi_v, si_v, ew_v, rows_v, agg_sh, sem):
        cid = lax.axis_index("c")
        sid = lax.axis_index("s")
        wid = cid * NS + sid
        base = wid * ew_per

        pltpu.sync_copy(zeros_hbm.at[pl.ds(sid * rt, rt)],
                        agg_sh.at[pl.ds(sid * rt, rt)])
        plsc.subcore_barrier()

        def chunk(j, carry):
            off = base + j * ch
            pltpu.sync_copy(gidx_hbm.at[pl.ds(off, ch)], gi_v)
            pltpu.sync_copy(sidx_hbm.at[pl.ds(off, ch)], si_v)
            pltpu.sync_copy(ew_hbm.at[pl.ds(off, ch)], ew_v)
            pltpu.async_copy(table_hbm.at[gi_v], rows_v, sem).wait()

            def scale(i, c2):
                ew16 = ew_v[pl.ds(i * 16, 16)]
                for k in range(16):
                    r = i * 16 + k
                    rows_v[r, :] = rows_v[r, :] * ew16[k]
                return c2

            lax.fori_loop(0, ch // 16, scale, 0)
            pltpu.sync_copy(rows_v, agg_sh.at[si_v], add=True)
            return carry

        lax.fori_loop(0, nchunk, chunk, 0)
        plsc.subcore_barrier()

        pltpu.sync_copy(agg_sh.at[pl.ds(sid * rt, rt)],
                        agg_out.at[cid, pl.ds(sid * rt, rt)])

    return mp_kernel(table, gidx, sidx, ew, zeros_r)


BR = 2048


def _row(c):
    return pl.BlockSpec((BR, c), lambda i: (i, 0))


def _rep(r, c):
    return pl.BlockSpec((r, c), lambda i: (0, 0))


def _tc_stage1(dvp_t, dcp_t, vc, vx, wve, bve, w2p, n_pad):
    def body(dv_ref, dc_ref, vc_ref, vx_ref, wve_ref, bve_ref, w2_ref,
             p_ref, nv_ref, nc_ref):
        nv = lax.rsqrt(jnp.maximum(dv_ref[:, 0:1] + dv_ref[:, 1:2], 1.0))
        nc = lax.rsqrt(jnp.maximum(dc_ref[:, 0:1] + dc_ref[:, 1:2], 1.0))
        x = jnp.maximum(
            vc_ref[...] * wve_ref[0:1, :] + vx_ref[...] * wve_ref[1:2, :]
            + bve_ref[...], 0.0)
        p_ref[...] = jnp.dot(x * nv, w2_ref[...],
                             preferred_element_type=jnp.float32)
        nv_ref[...] = nv
        nc_ref[...] = nc

    return pl.pallas_call(
        body,
        grid=(n_pad // BR,),
        in_specs=[_row(2), _row(2), _row(1), _row(1), _rep(2, 16),
                  _rep(1, 16), _rep(16, 16)],
        out_specs=(_row(16), _row(1), _row(1)),
        out_shape=(
            jax.ShapeDtypeStruct((n_pad, 16), jnp.float32),
            jax.ShapeDtypeStruct((n_pad, 1), jnp.float32),
            jax.ShapeDtypeStruct((n_pad, 1), jnp.float32),
        ),
    )(dvp_t, dcp_t, vc, vx, wve, bve, w2p)


def _tc_stage2(a0, a1, nc, b2p, w2p, n_pad):
    def body(a0_ref, a1_ref, nc_ref, b2_ref, w2_ref, q_ref):
        nc_col = nc_ref[...]
        h = jnp.maximum((a0_ref[...] + a1_ref[...]) * nc_col + b2_ref[...],
                        0.0)
        q_ref[...] = jnp.dot(h * nc_col, w2_ref[...],
                             preferred_element_type=jnp.float32)

    return pl.pallas_call(
        body,
        grid=(n_pad // BR,),
        in_specs=[_row(16), _row(16), _row(1), _rep(1, 16), _rep(16, 16)],
        out_specs=_row(16),
        out_shape=jax.ShapeDtypeStruct((n_pad, 16), jnp.float32),
    )(a0, a1, nc, b2p, w2p)


def _tc_stage3(a0, a1, nv, b2p, wo1p, bo1p, wo2p, bo2p, wo3p, bo3p,
               n_valid, n_pad):
    def body(a0_ref, a1_ref, nv_ref, b2_ref, wo1_ref, bo1_ref, wo2_ref,
             bo2_ref, wo3_ref, bo3_ref, out_ref):
        h = jnp.maximum((a0_ref[...] + a1_ref[...]) * nv_ref[...]
                        + b2_ref[...], 0.0)
        h = jnp.maximum(jnp.dot(h, wo1_ref[...],
                                preferred_element_type=jnp.float32)
                        + bo1_ref[...], 0.0)
        h = jnp.maximum(jnp.dot(h, wo2_ref[...],
                                preferred_element_type=jnp.float32)
                        + bo2_ref[...], 0.0)
        logit = jnp.dot(h, wo3_ref[...],
                        preferred_element_type=jnp.float32) + bo3_ref[...]
        i = pl.program_id(0)
        row = lax.broadcasted_iota(jnp.int32, (BR, 16), 0) + i * BR
        col = lax.broadcasted_iota(jnp.int32, (BR, 16), 1)
        keep = jnp.logical_and(row < n_valid, col == 0)
        s = jnp.sum(jnp.where(keep, logit, 0.0), axis=(0, 1), keepdims=True)

        @pl.when(i == 0)
        def _():
            out_ref[...] = jnp.zeros((1, 1), jnp.float32)

        out_ref[...] += s / float(n_valid)

    return pl.pallas_call(
        body,
        grid=(n_pad // BR,),
        in_specs=[_row(16), _row(16), _row(1), _rep(1, 16), _rep(16, 16),
                  _rep(1, 16), _rep(16, 16), _rep(1, 16), _rep(16, 16),
                  _rep(1, 16)],
        out_specs=pl.BlockSpec((1, 1), lambda i: (0, 0)),
        out_shape=jax.ShapeDtypeStruct((1, 1), jnp.float32),
    )(a0, a1, nv, b2p, wo1p, bo1p, wo2p, bo2p, wo3p, bo3p)


def kernel(var_c, var_x, con_b, edge_src, edge_dst, edge_weight,
           W_ve, b_ve, W_ce, b_ce, W1, b1, W2, b2,
           Wo1, bo1, Wo2, bo2, Wo3, bo3):
    f32 = jnp.float32
    n = var_c.shape[0]
    n_pad = -(-n // BR) * BR

    pad = n_pad - n
    vc = jnp.pad(var_c, (0, pad)).reshape(n_pad, 1)
    vx = jnp.pad(var_x, (0, pad)).reshape(n_pad, 1)
    wve = jnp.pad(W_ve, ((0, 0), (0, 6)))
    bve = jnp.pad(b_ve, (0, 6)).reshape(1, 16)
    w2p = jnp.pad(W2, ((0, 6), (0, 6)))
    b2p = jnp.pad(b2, (0, 6)).reshape(1, 16)
    wo1p = jnp.pad(Wo1, ((0, 6), (0, 6)))
    bo1p = jnp.pad(bo1, (0, 6)).reshape(1, 16)
    wo2p = jnp.pad(Wo2, ((0, 6), (0, 6)))
    bo2p = jnp.pad(bo2, (0, 6)).reshape(1, 16)
    wo3p = jnp.pad(Wo3, ((0, 6), (0, 15)))
    bo3p = jnp.pad(bo3, (0, 15)).reshape(1, 16)
    ones_ch = jnp.ones((_chunk_size(edge_src.shape[0] // NW),), f32)
    zeros_n = jnp.zeros((n_pad,), f32)
    zeros_r = jnp.zeros((n_pad, 16), f32)

    dvp, dcp = _sc_degrees(edge_src, edge_dst, ones_ch, zeros_n, n_pad)

    p_tab, nv, nc = _tc_stage1(dvp.T, dcp.T, vc, vx, wve, bve, w2p, n_pad)

    agg_c = _sc_message_pass(p_tab, edge_src, edge_dst, edge_weight,
                             zeros_r, n_pad)

    q_tab = _tc_stage2(agg_c[0], agg_c[1], nc, b2p, w2p, n_pad)

    agg_v = _sc_message_pass(q_tab, edge_dst, edge_src, edge_weight,
                             zeros_r, n_pad)

    return _tc_stage3(agg_v[0], agg_v[1], nv, b2p, wo1p, bo1p, wo2p, bo2p,
                      wo3p, bo3p, n, n_pad)


# speedup vs baseline: 33.0826x; 1.3624x over previous
import functools

import jax
import jax.numpy as jnp
from jax import lax
from jax.experimental import pallas as pl
from jax.experimental.pallas import tpu as pltpu
from jax.experimental.pallas import tpu_sc as plsc

NC = 2
NS = 16
NW = NC * NS


def _chunk_size(ew_per):
    for ch in range(min(2000, ew_per) // 16 * 16, 0, -16):
        if ew_per % ch == 0:
            return ch
    for ch in range(min(2000, ew_per), 0, -1):
        if ew_per % ch == 0:
            return ch
    return 1


def _deg_chunk_size(ew_per):
    for ch in range(min(10000, ew_per), 0, -1):
        if ew_per % ch == 0:
            return ch
    return 1


def _mesh():
    return plsc.VectorSubcoreMesh(
        core_axis_name="c", subcore_axis_name="s", num_cores=NC,
        num_subcores=NS)


def _sc_degrees(edge_src, edge_dst, ones_ch, zeros_n, n_pad):
    e = edge_src.shape[0]
    ew_per = e // NW
    ch = ones_ch.shape[0]
    nchunk = ew_per // ch
    rt = n_pad // NS

    @functools.partial(
        pl.kernel,
        out_type=(
            jax.ShapeDtypeStruct((NC, n_pad), jnp.float32),
            jax.ShapeDtypeStruct((NC, n_pad), jnp.float32),
        ),
        mesh=_mesh(),
        compiler_params=pltpu.CompilerParams(use_tc_tiling_on_sc=False),
        scratch_types=[
            [pltpu.VMEM((ch,), jnp.int32) for _ in range(2)],
            [pltpu.VMEM((ch,), jnp.int32) for _ in range(2)],
            pltpu.VMEM((ch,), jnp.float32),
            pltpu.VMEM_SHARED((n_pad,), jnp.float32),
            pltpu.VMEM_SHARED((n_pad,), jnp.float32),
            [pltpu.SemaphoreType.DMA for _ in range(8)],
        ],
    )
    def deg_kernel(src_hbm, dst_hbm, ones_hbm, zeros_hbm, dv_out, dc_out,
                   sv_b, sd_b, ones_v, dv_sh, dc_sh, sems):
        cid = lax.axis_index("c")
        sid = lax.axis_index("s")
        wid = cid * NS + sid
        base = wid * ew_per
        lsv, lsd, ssv, ssd = sems[0:2], sems[2:4], sems[4:6], sems[6:8]

        pltpu.sync_copy(zeros_hbm, dv_sh.at[pl.ds(sid * rt, rt)])
        pltpu.sync_copy(zeros_hbm, dc_sh.at[pl.ds(sid * rt, rt)])
        pltpu.sync_copy(ones_hbm, ones_v)
        plsc.subcore_barrier()

        def start_l(j):
            k = j % 2
            off = base + j * ch
            return (
                pltpu.async_copy(src_hbm.at[pl.ds(off, ch)], sv_b[k],
                                 lsv[k]),
                pltpu.async_copy(dst_hbm.at[pl.ds(off, ch)], sd_b[k],
                                 lsd[k]),
            )

        ld, sc = {}, {}
        ld[0] = start_l(0)
        for j in range(nchunk):
            k = j % 2
            if j + 1 < nchunk:
                if j - 1 >= 0:
                    sc[j - 1][0].wait()
                    sc[j - 1][1].wait()
                ld[j + 1] = start_l(j + 1)
            ld[j][0].wait()
            ld[j][1].wait()
            sc[j] = (
                pltpu.async_copy(ones_v, dv_sh.at[sv_b[k]], ssv[k],
                                 add=True),
                pltpu.async_copy(ones_v, dc_sh.at[sd_b[k]], ssd[k],
                                 add=True),
            )
        for j in range(max(0, nchunk - 2), nchunk):
            sc[j][0].wait()
            sc[j][1].wait()
        plsc.subcore_barrier()

        pltpu.sync_copy(dv_sh.at[pl.ds(sid * rt, rt)],
                        dv_out.at[cid, pl.ds(sid * rt, rt)])
        pltpu.sync_copy(dc_sh.at[pl.ds(sid * rt, rt)],
                        dc_out.at[cid, pl.ds(sid * rt, rt)])

    return deg_kernel(edge_src, edge_dst, ones_ch, zeros_n)


def _sc_message_pass(table, gidx, sidx, ew, zeros_r, n_pad):
    e = gidx.shape[0]
    ew_per = e // NW
    ch = _chunk_size(ew_per)
    nchunk = ew_per // ch
    rt = n_pad // NS

    @functools.partial(
        pl.kernel,
        out_type=jax.ShapeDtypeStruct((NC, n_pad, 16), jnp.float32),
        mesh=_mesh(),
        compiler_params=pltpu.CompilerParams(use_tc_tiling_on_sc=False),
        scratch_types=[
            [pltpu.VMEM((ch,), jnp.int32) for _ in range(2)],
            [pltpu.VMEM((ch,), jnp.int32) for _ in range(2)],
            [pltpu.VMEM((ch,), jnp.float32) for _ in range(2)],
            [pltpu.VMEM((ch, 16), jnp.float32) for _ in range(2)],
            pltpu.VMEM_SHARED((n_pad, 16), jnp.float32),
            [pltpu.SemaphoreType.DMA for _ in range(8)],
        ],
    )
    def mp_kernel(table_hbm, gidx_hbm, sidx_hbm, ew_hbm, zeros_hbm, agg_out,
                  gi_b, si_b, ew_b, rows_b, agg_sh, sems):
        cid = lax.axis_index("c")
        sid = lax.axis_index("s")
        wid = cid * NS + sid
        base = wid * ew_per
        lsem, gsem = sems[0:2], sems[2:4]
        ssem = (sems[4:6], sems[6:8])

        h0 = (ch // 2 + 15) // 16 * 16
        if h0 >= ch:
            h0 = ch
        halves = ((0, h0), (h0, ch - h0)) if ch > h0 else ((0, ch),)

        pltpu.sync_copy(zeros_hbm, agg_sh.at[pl.ds(sid * rt, rt)])
        plsc.subcore_barrier()

        def start_l(j):
            k = j % 2
            off = base + j * ch
            return (
                pltpu.async_copy(gidx_hbm.at[pl.ds(off, ch)], gi_b[k],
                                 lsem[k]),
                pltpu.async_copy(sidx_hbm.at[pl.ds(off, ch)], si_b[k],
                                 lsem[k]),
                pltpu.async_copy(ew_hbm.at[pl.ds(off, ch)], ew_b[k],
                                 lsem[k]),
            )

        def scale(k, lo, nrow):
            rows_v, ew_v = rows_b[k], ew_b[k]

            @plsc.parallel_loop(0, nrow // 16, 1, unroll=2)
            def _(i):
                ew16 = ew_v[pl.ds(lo + i * 16, 16)]
                for t in range(16):
                    r = lo + i * 16 + t
                    rows_v[r, :] = rows_v[r, :] * ew16[t]

        ld, gd, sd = {}, {}, {}
        ld[0] = start_l(0)
        for d in ld[0]:
            d.wait()
        gd[0] = pltpu.async_copy(table_hbm.at[gi_b[0]], rows_b[0], gsem[0])
        for j in range(nchunk):
            k, kn = j % 2, (j + 1) % 2
            if j + 1 < nchunk:
                if j - 1 >= 0:
                    for d in sd[j - 1]:
                        d.wait()
                ld[j + 1] = start_l(j + 1)
            gd[j].wait()
            if j + 1 < nchunk:
                for d in ld[j + 1]:
                    d.wait()
                gd[j + 1] = pltpu.async_copy(table_hbm.at[gi_b[kn]],
                                             rows_b[kn], gsem[kn])
            descs = []
            for hi, (lo, nrow) in enumerate(halves):
                scale(k, lo, nrow)
                descs.append(pltpu.async_copy(
                    rows_b[k].at[pl.ds(lo, nrow)],
                    agg_sh.at[si_b[k].at[pl.ds(lo, nrow)]],
                    ssem[k][hi], add=True))
            sd[j] = tuple(descs)
        for j in range(max(0, nchunk - 2), nchunk):
            for d in sd[j]:
                d.wait()
        plsc.subcore_barrier()

        pltpu.sync_copy(agg_sh.at[pl.ds(sid * rt, rt)],
                        agg_out.at[cid, pl.ds(sid * rt, rt)])

    return mp_kernel(table, gidx, sidx, ew, zeros_r)


BR = 2048


def _row(c):
    return pl.BlockSpec((BR, c), lambda i: (i, 0))


def _rep(r, c):
    return pl.BlockSpec((r, c), lambda i: (0, 0))


def _tc_stage1(dvp_t, dcp_t, vc, vx, wve, bve, w2p, n_pad):
    def body(dv_ref, dc_ref, vc_ref, vx_ref, wve_ref, bve_ref, w2_ref,
             p_ref, nv_ref, nc_ref):
        nv = lax.rsqrt(jnp.maximum(dv_ref[:, 0:1] + dv_ref[:, 1:2], 1.0))
        nc = lax.rsqrt(jnp.maximum(dc_ref[:, 0:1] + dc_ref[:, 1:2], 1.0))
        x = jnp.maximum(
            vc_ref[...] * wve_ref[0:1, :] + vx_ref[...] * wve_ref[1:2, :]
            + bve_ref[...], 0.0)
        p_ref[...] = jnp.dot(x * nv, w2_ref[...],
                             preferred_element_type=jnp.float32)
        nv_ref[...] = nv
        nc_ref[...] = nc

    return pl.pallas_call(
        body,
        grid=(n_pad // BR,),
        in_specs=[_row(2), _row(2), _row(1), _row(1), _rep(2, 16),
                  _rep(1, 16), _rep(16, 16)],
        out_specs=(_row(16), _row(1), _row(1)),
        out_shape=(
            jax.ShapeDtypeStruct((n_pad, 16), jnp.float32),
            jax.ShapeDtypeStruct((n_pad, 1), jnp.float32),
            jax.ShapeDtypeStruct((n_pad, 1), jnp.float32),
        ),
    )(dvp_t, dcp_t, vc, vx, wve, bve, w2p)


def _tc_stage2(a0, a1, nc, b2p, w2p, n_pad):
    def body(a0_ref, a1_ref, nc_ref, b2_ref, w2_ref, q_ref):
        nc_col = nc_ref[...]
        h = jnp.maximum((a0_ref[...] + a1_ref[...]) * nc_col + b2_ref[...],
                        0.0)
        q_ref[...] = jnp.dot(h * nc_col, w2_ref[...],
                             preferred_element_type=jnp.float32)

    return pl.pallas_call(
        body,
        grid=(n_pad // BR,),
        in_specs=[_row(16), _row(16), _row(1), _rep(1, 16), _rep(16, 16)],
        out_specs=_row(16),
        out_shape=jax.ShapeDtypeStruct((n_pad, 16), jnp.float32),
    )(a0, a1, nc, b2p, w2p)


def _tc_stage3(a0, a1, nv, b2p, wo1p, bo1p, wo2p, bo2p, wo3p, bo3p,
               n_valid, n_pad):
    def body(a0_ref, a1_ref, nv_ref, b2_ref, wo1_ref, bo1_ref, wo2_ref,
             bo2_ref, wo3_ref, bo3_ref, out_ref):
        h = jnp.maximum((a0_ref[...] + a1_ref[...]) * nv_ref[...]
                        + b2_ref[...], 0.0)
        h = jnp.maximum(jnp.dot(h, wo1_ref[...],
                                preferred_element_type=jnp.float32)
                        + bo1_ref[...], 0.0)
        h = jnp.maximum(jnp.dot(h, wo2_ref[...],
                                preferred_element_type=jnp.float32)
                        + bo2_ref[...], 0.0)
        logit = jnp.dot(h, wo3_ref[...],
                        preferred_element_type=jnp.float32) + bo3_ref[...]
        i = pl.program_id(0)
        row = lax.broadcasted_iota(jnp.int32, (BR, 16), 0) + i * BR
        col = lax.broadcasted_iota(jnp.int32, (BR, 16), 1)
        keep = jnp.logical_and(row < n_valid, col == 0)
        s = jnp.sum(jnp.where(keep, logit, 0.0), axis=(0, 1), keepdims=True)

        @pl.when(i == 0)
        def _():
            out_ref[...] = jnp.zeros((1, 1), jnp.float32)

        out_ref[...] += s / float(n_valid)

    return pl.pallas_call(
        body,
        grid=(n_pad // BR,),
        in_specs=[_row(16), _row(16), _row(1), _rep(1, 16), _rep(16, 16),
                  _rep(1, 16), _rep(16, 16), _rep(1, 16), _rep(16, 16),
                  _rep(1, 16)],
        out_specs=pl.BlockSpec((1, 1), lambda i: (0, 0)),
        out_shape=jax.ShapeDtypeStruct((1, 1), jnp.float32),
    )(a0, a1, nv, b2p, wo1p, bo1p, wo2p, bo2p, wo3p, bo3p)


def kernel(var_c, var_x, con_b, edge_src, edge_dst, edge_weight,
           W_ve, b_ve, W_ce, b_ce, W1, b1, W2, b2,
           Wo1, bo1, Wo2, bo2, Wo3, bo3):
    f32 = jnp.float32
    n = var_c.shape[0]
    n_pad = -(-n // BR) * BR

    pad = n_pad - n
    vc = jnp.pad(var_c, (0, pad)).reshape(n_pad, 1)
    vx = jnp.pad(var_x, (0, pad)).reshape(n_pad, 1)
    wve = jnp.pad(W_ve, ((0, 0), (0, 6)))
    bve = jnp.pad(b_ve, (0, 6)).reshape(1, 16)
    w2p = jnp.pad(W2, ((0, 6), (0, 6)))
    b2p = jnp.pad(b2, (0, 6)).reshape(1, 16)
    wo1p = jnp.pad(Wo1, ((0, 6), (0, 6)))
    bo1p = jnp.pad(bo1, (0, 6)).reshape(1, 16)
    wo2p = jnp.pad(Wo2, ((0, 6), (0, 6)))
    bo2p = jnp.pad(bo2, (0, 6)).reshape(1, 16)
    wo3p = jnp.pad(Wo3, ((0, 6), (0, 15)))
    bo3p = jnp.pad(bo3, (0, 15)).reshape(1, 16)
    ones_ch = jnp.ones((_deg_chunk_size(edge_src.shape[0] // NW),), f32)
    zeros_n = jnp.zeros((n_pad // NS,), f32)
    zeros_r = jnp.zeros((n_pad // NS, 16), f32)

    dvp, dcp = _sc_degrees(edge_src, edge_dst, ones_ch, zeros_n, n_pad)

    p_tab, nv, nc = _tc_stage1(dvp.T, dcp.T, vc, vx, wve, bve, w2p, n_pad)

    agg_c = _sc_message_pass(p_tab, edge_src, edge_dst, edge_weight,
                             zeros_r, n_pad)

    q_tab = _tc_stage2(agg_c[0], agg_c[1], nc, b2p, w2p, n_pad)

    agg_v = _sc_message_pass(q_tab, edge_dst, edge_src, edge_weight,
                             zeros_r, n_pad)

    return _tc_stage3(agg_v[0], agg_v[1], nv, b2p, wo1p, bo1p, wo2p, bo2p,
                      wo3p, bo3p, n, n_pad)
